# baseline (speedup 1.0000x reference)
import functools

import jax
import jax.numpy as jnp
from jax import lax
from jax.experimental import pallas as pl
from jax.experimental.pallas import tpu as pltpu
from jax.experimental.pallas import tpu_sc as plsc

N_NODES = 10000
D = 128

NC = 2
NS = 16
NW = NC * NS
CHUNK = 128
N_CHUNKS = 80
HALF = N_CHUNKS // 2
C0_STAGES = 1
C1_STAGES = 3
E0 = NS * C0_STAGES * HALF * CHUNK
E1 = NS * C1_STAGES * HALF * CHUNK
E_PAD = NW * N_CHUNKS * CHUNK

ACC_ROWS = 10240
ZROWS = ACC_ROWS // NS

_mesh = plsc.VectorSubcoreMesh(core_axis_name="c", subcore_axis_name="s")


def _sc_count_body(dst_hbm, zeros_hbm, ones_hbm, out_hbm, dst_v, ones_v, acc):
  c = lax.axis_index("c")
  s = lax.axis_index("s")
  wid = s * NC + c
  pltpu.sync_copy(dst_hbm.at[wid], dst_v)
  pltpu.sync_copy(ones_hbm, ones_v)
  pltpu.sync_copy(zeros_hbm, acc.at[pl.ds(s * ZROWS, ZROWS)])
  plsc.subcore_barrier()

  @pl.loop(0, N_CHUNKS)
  def _(j):
    pltpu.sync_copy(ones_v, acc.at[dst_v.at[j]], add=True)

  plsc.subcore_barrier()
  pltpu.sync_copy(acc.at[pl.ds(s * ZROWS, ZROWS)],
                  out_hbm.at[c, pl.ds(s * ZROWS, ZROWS)])


_sc_count = pl.kernel(
    _sc_count_body,
    out_type=jax.ShapeDtypeStruct((NC, ACC_ROWS, D), jnp.float32),
    mesh=_mesh,
    scratch_types=[
        pltpu.VMEM((N_CHUNKS, CHUNK), jnp.int32),
        pltpu.VMEM((CHUNK, D), jnp.float32),
        pltpu.VMEM_SHARED((ACC_ROWS, D), jnp.float32),
    ],
)


def _emit_edge_pipeline(g_hbm, src_hbm, dst_hbm, s, n_stages, acc,
                        src_v, dst_v, buf0, buf1,
                        sem_g0, sem_g1, sem_s0, sem_s1):
  for h in range(n_stages):
    pltpu.sync_copy(src_hbm.at[s, pl.ds(h * HALF, HALF)], src_v)
    pltpu.sync_copy(dst_hbm.at[s, pl.ds(h * HALF, HALF)], dst_v)
    pltpu.async_copy(g_hbm.at[src_v.at[0]], buf0, sem_g0)
    pltpu.async_copy(g_hbm.at[src_v.at[1]], buf1, sem_g1)

    @pl.loop(0, HALF // 2 - 1)
    def _(i):
      j0 = 2 * i
      j1 = j0 + 1
      pltpu.make_async_copy(g_hbm.at[src_v.at[j0]], buf0, sem_g0).wait()
      pltpu.async_copy(buf0, acc.at[dst_v.at[j0]], sem_s0, add=True)
      pltpu.make_async_copy(g_hbm.at[src_v.at[j1]], buf1, sem_g1).wait()
      pltpu.async_copy(buf1, acc.at[dst_v.at[j1]], sem_s1, add=True)
      pltpu.make_async_copy(buf0, acc.at[dst_v.at[j0]], sem_s0).wait()
      pltpu.async_copy(g_hbm.at[src_v.at[j0 + 2]], buf0, sem_g0)
      pltpu.make_async_copy(buf1, acc.at[dst_v.at[j1]], sem_s1).wait()
      pltpu.async_copy(g_hbm.at[src_v.at[j1 + 2]], buf1, sem_g1)

    jl0 = HALF - 2
    jl1 = HALF - 1
    pltpu.make_async_copy(g_hbm.at[src_v.at[jl0]], buf0, sem_g0).wait()
    pltpu.async_copy(buf0, acc.at[dst_v.at[jl0]], sem_s0, add=True)
    pltpu.make_async_copy(g_hbm.at[src_v.at[jl1]], buf1, sem_g1).wait()
    pltpu.async_copy(buf1, acc.at[dst_v.at[jl1]], sem_s1, add=True)
    pltpu.make_async_copy(buf0, acc.at[dst_v.at[jl0]], sem_s0).wait()
    pltpu.make_async_copy(buf1, acc.at[dst_v.at[jl1]], sem_s1).wait()


def _sc_segsum_body(g_hbm, src0_hbm, dst0_hbm, src1_hbm, dst1_hbm,
                    zeros_hbm, out_hbm,
                    src_v, dst_v, buf0, buf1, acc,
                    sem_g0, sem_g1, sem_s0, sem_s1):
  c = lax.axis_index("c")
  s = lax.axis_index("s")
  pltpu.sync_copy(zeros_hbm, acc.at[pl.ds(s * ZROWS, ZROWS)])
  plsc.subcore_barrier()

  @pl.when(c == 0)
  def _():
    _emit_edge_pipeline(g_hbm, src0_hbm, dst0_hbm, s, C0_STAGES, acc,
                        src_v, dst_v, buf0, buf1,
                        sem_g0, sem_g1, sem_s0, sem_s1)

  @pl.when(c == 1)
  def _():
    _emit_edge_pipeline(g_hbm, src1_hbm, dst1_hbm, s, C1_STAGES, acc,
                        src_v, dst_v, buf0, buf1,
                        sem_g0, sem_g1, sem_s0, sem_s1)

  plsc.subcore_barrier()
  pltpu.sync_copy(acc.at[pl.ds(s * ZROWS, ZROWS)],
                  out_hbm.at[c, pl.ds(s * ZROWS, ZROWS)])


_sc_segsum = pl.kernel(
    _sc_segsum_body,
    out_type=jax.ShapeDtypeStruct((NC, ACC_ROWS, D), jnp.float32),
    mesh=_mesh,
    scratch_types=[
        pltpu.VMEM((HALF, CHUNK), jnp.int32),
        pltpu.VMEM((HALF, CHUNK), jnp.int32),
        pltpu.VMEM((CHUNK, D), jnp.float32),
        pltpu.VMEM((CHUNK, D), jnp.float32),
        pltpu.VMEM_SHARED((ACC_ROWS, D), jnp.float32),
        pltpu.SemaphoreType.DMA,
        pltpu.SemaphoreType.DMA,
        pltpu.SemaphoreType.DMA,
        pltpu.SemaphoreType.DMA,
    ],
)


_ROWS = 1000
_GRID = N_NODES // _ROWS


def _dis_from_cnt(cnt):
  indeg = cnt[0, :, :1] + cnt[1, :, :1]
  return lax.rsqrt(indeg + 1.0)


def _tc1a_body(x_ref, w_ref, h_ref):
  h_ref[...] = jnp.dot(x_ref[...], w_ref[...],
                       preferred_element_type=jnp.float32)


def _tc1b_body(cnt_ref, h_ref, g_ref):
  g_ref[...] = h_ref[...] * _dis_from_cnt(cnt_ref[...])


def _tc2_body(cnt_ref, s_ref, g_ref, b_ref, w_ref, o_ref):
  dis = _dis_from_cnt(cnt_ref[...])
  tot = s_ref[0] + s_ref[1] + g_ref[...]
  pre = dis * tot + b_ref[...]
  z = 1.0507009873554805 * jnp.where(
      pre > 0, pre, 1.6732632423543772 * (jnp.exp(jnp.minimum(pre, 0.0)) - 1.0))
  h2 = jnp.dot(z, w_ref[...], preferred_element_type=jnp.float32)
  o_ref[...] = h2 * dis


def _tc3_body(cnt_ref, s_ref, g_ref, b_ref, o_ref):
  dis = _dis_from_cnt(cnt_ref[...])
  tot = s_ref[0] + s_ref[1] + g_ref[...]
  o = dis * tot + b_ref[...]
  m = jnp.max(o, axis=1, keepdims=True)
  e = o - m
  lse = jnp.log(jnp.sum(jnp.exp(e), axis=1, keepdims=True))
  o_ref[...] = e - lse


_cnt_spec = pl.BlockSpec((NC, _ROWS, D), lambda i: (0, i, 0))
_row_spec = pl.BlockSpec((_ROWS, D), lambda i: (i, 0))
_par_spec = pl.BlockSpec((NC, _ROWS, D), lambda i: (0, i, 0))
_w_spec = pl.BlockSpec((D, D), lambda i: (0, 0))
_b_spec = pl.BlockSpec((1, D), lambda i: (0, 0))
_out_sds = jax.ShapeDtypeStruct((N_NODES, D), jnp.float32)

_tc1a = pl.pallas_call(
    _tc1a_body, grid=(_GRID,),
    in_specs=[_row_spec, _w_spec],
    out_specs=_row_spec, out_shape=_out_sds)

_tc1b = pl.pallas_call(
    _tc1b_body, grid=(_GRID,),
    in_specs=[_cnt_spec, _row_spec],
    out_specs=_row_spec, out_shape=_out_sds)

_tc2 = pl.pallas_call(
    _tc2_body, grid=(_GRID,),
    in_specs=[_cnt_spec, _par_spec, _row_spec, _b_spec, _w_spec],
    out_specs=_row_spec, out_shape=_out_sds)

_tc3 = pl.pallas_call(
    _tc3_body, grid=(_GRID,),
    in_specs=[_cnt_spec, _par_spec, _row_spec, _b_spec],
    out_specs=_row_spec, out_shape=_out_sds)


@jax.jit
def _run(x, edge_index, W1, b1, W2, b2):
  src = edge_index[0].astype(jnp.int32)
  dst = edge_index[1].astype(jnp.int32)
  n_edges = src.shape[0]
  pad = E_PAD - n_edges
  src_f = jnp.concatenate([src, jnp.zeros((pad,), jnp.int32)])
  dst_f = jnp.concatenate([dst, jnp.full((pad,), N_NODES, jnp.int32)])
  dst_p = dst_f.reshape(NW, N_CHUNKS, CHUNK)
  src0 = src_f[:E0].reshape(NS, C0_STAGES * HALF, CHUNK)
  dst0 = dst_f[:E0].reshape(NS, C0_STAGES * HALF, CHUNK)
  src1 = src_f[E0:].reshape(NS, C1_STAGES * HALF, CHUNK)
  dst1 = dst_f[E0:].reshape(NS, C1_STAGES * HALF, CHUNK)

  zerosD = jnp.zeros((ZROWS, D), jnp.float32)
  onesD = jnp.ones((CHUNK, D), jnp.float32)

  cnt = _sc_count(dst_p, zerosD, onesD)
  h1 = _tc1a(x, W1)
  g1 = _tc1b(cnt, h1)
  s1 = _sc_segsum(g1, src0, dst0, src1, dst1, zerosD)
  g2 = _tc2(cnt, s1, g1, b1.reshape(1, D), W2)
  s2 = _sc_segsum(g2, src0, dst0, src1, dst1, zerosD)
  return _tc3(cnt, s2, g2, b2.reshape(1, D))


def kernel(x, edge_index, W1, b1, W2, b2):
  return _run(x, edge_index, W1, b1, W2, b2)


# speedup vs baseline: 1.1618x; 1.1618x over previous
import functools

import jax
import jax.numpy as jnp
from jax import lax
from jax.experimental import pallas as pl
from jax.experimental.pallas import tpu as pltpu
from jax.experimental.pallas import tpu_sc as plsc

N_NODES = 10000
D = 128

NC = 2
NS = 16
NW = NC * NS
CHUNK = 128
N_CHUNKS = 80
HALF = N_CHUNKS // 2
C0_STAGES = 3
C1_STAGES = 1
E0 = NS * C0_STAGES * HALF * CHUNK
E1 = NS * C1_STAGES * HALF * CHUNK
E_PAD = NW * N_CHUNKS * CHUNK

ACC_ROWS = 10240
ZROWS = ACC_ROWS // NS

_mesh = plsc.VectorSubcoreMesh(core_axis_name="c", subcore_axis_name="s")


def _sc_count_body(dst_hbm, zeros_hbm, ones_hbm, out_hbm, dst_v, ones_v, acc):
  c = lax.axis_index("c")
  s = lax.axis_index("s")
  wid = s * NC + c
  pltpu.sync_copy(dst_hbm.at[wid], dst_v)
  pltpu.sync_copy(ones_hbm, ones_v)
  pltpu.sync_copy(zeros_hbm, acc.at[pl.ds(s * ZROWS, ZROWS)])
  plsc.subcore_barrier()

  @pl.loop(0, N_CHUNKS)
  def _(j):
    pltpu.sync_copy(ones_v, acc.at[dst_v.at[j]], add=True)

  plsc.subcore_barrier()
  pltpu.sync_copy(acc.at[pl.ds(s * ZROWS, ZROWS)],
                  out_hbm.at[c, pl.ds(s * ZROWS, ZROWS)])


_sc_count = pl.kernel(
    _sc_count_body,
    out_type=jax.ShapeDtypeStruct((NC, ACC_ROWS, D), jnp.float32),
    mesh=_mesh,
    scratch_types=[
        pltpu.VMEM((N_CHUNKS, CHUNK), jnp.int32),
        pltpu.VMEM((CHUNK, D), jnp.float32),
        pltpu.VMEM_SHARED((ACC_ROWS, D), jnp.float32),
    ],
)


def _emit_edge_pipeline(g_hbm, src_hbm, dst_hbm, s, n_stages, acc,
                        src_v, dst_v, buf0, buf1,
                        sem_g0, sem_g1, sem_s0, sem_s1):
  for h in range(n_stages):
    pltpu.sync_copy(src_hbm.at[s, pl.ds(h * HALF, HALF)], src_v)
    pltpu.sync_copy(dst_hbm.at[s, pl.ds(h * HALF, HALF)], dst_v)
    pltpu.async_copy(g_hbm.at[src_v.at[0]], buf0, sem_g0)
    pltpu.async_copy(g_hbm.at[src_v.at[1]], buf1, sem_g1)

    @pl.loop(0, HALF // 2 - 1)
    def _(i):
      j0 = 2 * i
      j1 = j0 + 1
      pltpu.make_async_copy(g_hbm.at[src_v.at[j0]], buf0, sem_g0).wait()
      pltpu.async_copy(buf0, acc.at[dst_v.at[j0]], sem_s0, add=True)
      pltpu.make_async_copy(g_hbm.at[src_v.at[j1]], buf1, sem_g1).wait()
      pltpu.async_copy(buf1, acc.at[dst_v.at[j1]], sem_s1, add=True)
      pltpu.make_async_copy(buf0, acc.at[dst_v.at[j0]], sem_s0).wait()
      pltpu.async_copy(g_hbm.at[src_v.at[j0 + 2]], buf0, sem_g0)
      pltpu.make_async_copy(buf1, acc.at[dst_v.at[j1]], sem_s1).wait()
      pltpu.async_copy(g_hbm.at[src_v.at[j1 + 2]], buf1, sem_g1)

    jl0 = HALF - 2
    jl1 = HALF - 1
    pltpu.make_async_copy(g_hbm.at[src_v.at[jl0]], buf0, sem_g0).wait()
    pltpu.async_copy(buf0, acc.at[dst_v.at[jl0]], sem_s0, add=True)
    pltpu.make_async_copy(g_hbm.at[src_v.at[jl1]], buf1, sem_g1).wait()
    pltpu.async_copy(buf1, acc.at[dst_v.at[jl1]], sem_s1, add=True)
    pltpu.make_async_copy(buf0, acc.at[dst_v.at[jl0]], sem_s0).wait()
    pltpu.make_async_copy(buf1, acc.at[dst_v.at[jl1]], sem_s1).wait()


def _sc_segsum_body(g_hbm, src0_hbm, dst0_hbm, src1_hbm, dst1_hbm,
                    zeros_hbm, out_hbm,
                    src_v, dst_v, buf0, buf1, acc,
                    sem_g0, sem_g1, sem_s0, sem_s1):
  c = lax.axis_index("c")
  s = lax.axis_index("s")
  pltpu.sync_copy(zeros_hbm, acc.at[pl.ds(s * ZROWS, ZROWS)])
  plsc.subcore_barrier()

  @pl.when(c == 0)
  def _():
    _emit_edge_pipeline(g_hbm, src0_hbm, dst0_hbm, s, C0_STAGES, acc,
                        src_v, dst_v, buf0, buf1,
                        sem_g0, sem_g1, sem_s0, sem_s1)

  @pl.when(c == 1)
  def _():
    _emit_edge_pipeline(g_hbm, src1_hbm, dst1_hbm, s, C1_STAGES, acc,
                        src_v, dst_v, buf0, buf1,
                        sem_g0, sem_g1, sem_s0, sem_s1)

  plsc.subcore_barrier()
  pltpu.sync_copy(acc.at[pl.ds(s * ZROWS, ZROWS)],
                  out_hbm.at[c, pl.ds(s * ZROWS, ZROWS)])


_sc_segsum = pl.kernel(
    _sc_segsum_body,
    out_type=jax.ShapeDtypeStruct((NC, ACC_ROWS, D), jnp.float32),
    mesh=_mesh,
    scratch_types=[
        pltpu.VMEM((HALF, CHUNK), jnp.int32),
        pltpu.VMEM((HALF, CHUNK), jnp.int32),
        pltpu.VMEM((CHUNK, D), jnp.float32),
        pltpu.VMEM((CHUNK, D), jnp.float32),
        pltpu.VMEM_SHARED((ACC_ROWS, D), jnp.float32),
        pltpu.SemaphoreType.DMA,
        pltpu.SemaphoreType.DMA,
        pltpu.SemaphoreType.DMA,
        pltpu.SemaphoreType.DMA,
    ],
)


_ROWS = 1000
_GRID = N_NODES // _ROWS


def _dis_from_cnt(cnt):
  indeg = cnt[0, :, :1] + cnt[1, :, :1]
  return lax.rsqrt(indeg + 1.0)


def _tc1a_body(x_ref, w_ref, h_ref):
  h_ref[...] = jnp.dot(x_ref[...], w_ref[...],
                       preferred_element_type=jnp.float32)


def _tc1b_body(cnt_ref, h_ref, g_ref):
  g_ref[...] = h_ref[...] * _dis_from_cnt(cnt_ref[...])


def _tc2_body(cnt_ref, s_ref, g_ref, b_ref, w_ref, o_ref):
  dis = _dis_from_cnt(cnt_ref[...])
  tot = s_ref[0] + s_ref[1] + g_ref[...]
  pre = dis * tot + b_ref[...]
  z = 1.0507009873554805 * jnp.where(
      pre > 0, pre, 1.6732632423543772 * (jnp.exp(jnp.minimum(pre, 0.0)) - 1.0))
  h2 = jnp.dot(z, w_ref[...], preferred_element_type=jnp.float32)
  o_ref[...] = h2 * dis


def _tc3_body(cnt_ref, s_ref, g_ref, b_ref, o_ref):
  dis = _dis_from_cnt(cnt_ref[...])
  tot = s_ref[0] + s_ref[1] + g_ref[...]
  o = dis * tot + b_ref[...]
  m = jnp.max(o, axis=1, keepdims=True)
  e = o - m
  lse = jnp.log(jnp.sum(jnp.exp(e), axis=1, keepdims=True))
  o_ref[...] = e - lse


_cnt_spec = pl.BlockSpec((NC, _ROWS, D), lambda i: (0, i, 0))
_row_spec = pl.BlockSpec((_ROWS, D), lambda i: (i, 0))
_par_spec = pl.BlockSpec((NC, _ROWS, D), lambda i: (0, i, 0))
_w_spec = pl.BlockSpec((D, D), lambda i: (0, 0))
_b_spec = pl.BlockSpec((1, D), lambda i: (0, 0))
_out_sds = jax.ShapeDtypeStruct((N_NODES, D), jnp.float32)

_tc1a = pl.pallas_call(
    _tc1a_body, grid=(_GRID,),
    in_specs=[_row_spec, _w_spec],
    out_specs=_row_spec, out_shape=_out_sds)

_tc1b = pl.pallas_call(
    _tc1b_body, grid=(_GRID,),
    in_specs=[_cnt_spec, _row_spec],
    out_specs=_row_spec, out_shape=_out_sds)

_tc2 = pl.pallas_call(
    _tc2_body, grid=(_GRID,),
    in_specs=[_cnt_spec, _par_spec, _row_spec, _b_spec, _w_spec],
    out_specs=_row_spec, out_shape=_out_sds)

_tc3 = pl.pallas_call(
    _tc3_body, grid=(_GRID,),
    in_specs=[_cnt_spec, _par_spec, _row_spec, _b_spec],
    out_specs=_row_spec, out_shape=_out_sds)


@jax.jit
def _run(x, edge_index, W1, b1, W2, b2):
  src = edge_index[0].astype(jnp.int32)
  dst = edge_index[1].astype(jnp.int32)
  n_edges = src.shape[0]
  pad = E_PAD - n_edges
  src_f = jnp.concatenate([src, jnp.zeros((pad,), jnp.int32)])
  dst_f = jnp.concatenate([dst, jnp.full((pad,), N_NODES, jnp.int32)])
  dst_p = dst_f.reshape(NW, N_CHUNKS, CHUNK)
  src0 = src_f[:E0].reshape(NS, C0_STAGES * HALF, CHUNK)
  dst0 = dst_f[:E0].reshape(NS, C0_STAGES * HALF, CHUNK)
  src1 = src_f[E0:].reshape(NS, C1_STAGES * HALF, CHUNK)
  dst1 = dst_f[E0:].reshape(NS, C1_STAGES * HALF, CHUNK)

  zerosD = jnp.zeros((ZROWS, D), jnp.float32)
  onesD = jnp.ones((CHUNK, D), jnp.float32)

  cnt = _sc_count(dst_p, zerosD, onesD)
  h1 = _tc1a(x, W1)
  g1 = _tc1b(cnt, h1)
  s1 = _sc_segsum(g1, src0, dst0, src1, dst1, zerosD)
  g2 = _tc2(cnt, s1, g1, b1.reshape(1, D), W2)
  s2 = _sc_segsum(g2, src0, dst0, src1, dst1, zerosD)
  return _tc3(cnt, s2, g2, b2.reshape(1, D))


def kernel(x, edge_index, W1, b1, W2, b2):
  return _run(x, edge_index, W1, b1, W2, b2)


# speedup vs baseline: 1.3098x; 1.1274x over previous
import functools

import jax
import jax.numpy as jnp
from jax import lax
from jax.experimental import pallas as pl
from jax.experimental.pallas import tpu as pltpu
from jax.experimental.pallas import tpu_sc as plsc

N_NODES = 10000
D = 128

NC = 2
NS = 16
NW = NC * NS
CHUNK = 128
N_CHUNKS = 80
HALF = N_CHUNKS // 2
C0_STAGES = (40, 40, 40, 20)
C1_STAGES = (20,)
E0 = NS * sum(C0_STAGES) * CHUNK
E1 = NS * sum(C1_STAGES) * CHUNK
E_PAD = NW * N_CHUNKS * CHUNK

ACC_ROWS = 10240
ZROWS = ACC_ROWS // NS

_mesh = plsc.VectorSubcoreMesh(core_axis_name="c", subcore_axis_name="s")


def _sc_count_body(dst_hbm, zeros_hbm, ones_hbm, out_hbm, dst_v, ones_v, acc):
  c = lax.axis_index("c")
  s = lax.axis_index("s")
  wid = s * NC + c
  pltpu.sync_copy(dst_hbm.at[wid], dst_v)
  pltpu.sync_copy(ones_hbm, ones_v)
  pltpu.sync_copy(zeros_hbm, acc.at[pl.ds(s * ZROWS, ZROWS)])
  plsc.subcore_barrier()

  @pl.loop(0, N_CHUNKS)
  def _(j):
    pltpu.sync_copy(ones_v, acc.at[dst_v.at[j]], add=True)

  plsc.subcore_barrier()
  pltpu.sync_copy(acc.at[pl.ds(s * ZROWS, ZROWS)],
                  out_hbm.at[c, pl.ds(s * ZROWS, ZROWS)])


_sc_count = pl.kernel(
    _sc_count_body,
    out_type=jax.ShapeDtypeStruct((NC, ACC_ROWS, D), jnp.float32),
    mesh=_mesh,
    scratch_types=[
        pltpu.VMEM((N_CHUNKS, CHUNK), jnp.int32),
        pltpu.VMEM((CHUNK, D), jnp.float32),
        pltpu.VMEM_SHARED((ACC_ROWS, D), jnp.float32),
    ],
)


def _emit_edge_pipeline(g_hbm, src_hbm, dst_hbm, s, stages, acc,
                        src_v, dst_v, buf0, buf1,
                        sem_g0, sem_g1, sem_s0, sem_s1):
  base = 0
  for sz in stages:
    pltpu.sync_copy(src_hbm.at[s, pl.ds(base, sz)], src_v.at[pl.ds(0, sz)])
    pltpu.sync_copy(dst_hbm.at[s, pl.ds(base, sz)], dst_v.at[pl.ds(0, sz)])
    base += sz
    pltpu.async_copy(g_hbm.at[src_v.at[0]], buf0, sem_g0)
    pltpu.async_copy(g_hbm.at[src_v.at[1]], buf1, sem_g1)

    @pl.loop(0, sz // 2 - 1)
    def _(i):
      j0 = 2 * i
      j1 = j0 + 1
      pltpu.make_async_copy(g_hbm.at[src_v.at[j0]], buf0, sem_g0).wait()
      pltpu.async_copy(buf0, acc.at[dst_v.at[j0]], sem_s0, add=True)
      pltpu.make_async_copy(g_hbm.at[src_v.at[j1]], buf1, sem_g1).wait()
      pltpu.async_copy(buf1, acc.at[dst_v.at[j1]], sem_s1, add=True)
      pltpu.make_async_copy(buf0, acc.at[dst_v.at[j0]], sem_s0).wait()
      pltpu.async_copy(g_hbm.at[src_v.at[j0 + 2]], buf0, sem_g0)
      pltpu.make_async_copy(buf1, acc.at[dst_v.at[j1]], sem_s1).wait()
      pltpu.async_copy(g_hbm.at[src_v.at[j1 + 2]], buf1, sem_g1)

    jl0 = sz - 2
    jl1 = sz - 1
    pltpu.make_async_copy(g_hbm.at[src_v.at[jl0]], buf0, sem_g0).wait()
    pltpu.async_copy(buf0, acc.at[dst_v.at[jl0]], sem_s0, add=True)
    pltpu.make_async_copy(g_hbm.at[src_v.at[jl1]], buf1, sem_g1).wait()
    pltpu.async_copy(buf1, acc.at[dst_v.at[jl1]], sem_s1, add=True)
    pltpu.make_async_copy(buf0, acc.at[dst_v.at[jl0]], sem_s0).wait()
    pltpu.make_async_copy(buf1, acc.at[dst_v.at[jl1]], sem_s1).wait()


def _sc_segsum_body(g_hbm, src0_hbm, dst0_hbm, src1_hbm, dst1_hbm,
                    zeros_hbm, out_hbm,
                    src_v, dst_v, buf0, buf1, acc,
                    sem_g0, sem_g1, sem_s0, sem_s1):
  c = lax.axis_index("c")
  s = lax.axis_index("s")
  pltpu.sync_copy(zeros_hbm, acc.at[pl.ds(s * ZROWS, ZROWS)])
  plsc.subcore_barrier()

  @pl.when(c == 0)
  def _():
    _emit_edge_pipeline(g_hbm, src0_hbm, dst0_hbm, s, C0_STAGES, acc,
                        src_v, dst_v, buf0, buf1,
                        sem_g0, sem_g1, sem_s0, sem_s1)

  @pl.when(c == 1)
  def _():
    _emit_edge_pipeline(g_hbm, src1_hbm, dst1_hbm, s, C1_STAGES, acc,
                        src_v, dst_v, buf0, buf1,
                        sem_g0, sem_g1, sem_s0, sem_s1)

  plsc.subcore_barrier()
  pltpu.sync_copy(acc.at[pl.ds(s * ZROWS, ZROWS)],
                  out_hbm.at[c, pl.ds(s * ZROWS, ZROWS)])


_sc_segsum = pl.kernel(
    _sc_segsum_body,
    out_type=jax.ShapeDtypeStruct((NC, ACC_ROWS, D), jnp.float32),
    mesh=_mesh,
    scratch_types=[
        pltpu.VMEM((HALF, CHUNK), jnp.int32),
        pltpu.VMEM((HALF, CHUNK), jnp.int32),
        pltpu.VMEM((CHUNK, D), jnp.float32),
        pltpu.VMEM((CHUNK, D), jnp.float32),
        pltpu.VMEM_SHARED((ACC_ROWS, D), jnp.float32),
        pltpu.SemaphoreType.DMA,
        pltpu.SemaphoreType.DMA,
        pltpu.SemaphoreType.DMA,
        pltpu.SemaphoreType.DMA,
    ],
)


_ROWS = 1000
_GRID = N_NODES // _ROWS


def _dis_from_cnt(cnt):
  indeg = cnt[0, :, :1] + cnt[1, :, :1]
  return lax.rsqrt(indeg + 1.0)


def _tc1a_body(x_ref, w_ref, h_ref):
  h_ref[...] = jnp.dot(x_ref[...], w_ref[...],
                       preferred_element_type=jnp.float32)


def _tc1b_body(cnt_ref, h_ref, g_ref):
  g_ref[...] = h_ref[...] * _dis_from_cnt(cnt_ref[...])


def _tc2_body(cnt_ref, s_ref, g_ref, b_ref, w_ref, o_ref):
  dis = _dis_from_cnt(cnt_ref[...])
  tot = s_ref[0] + s_ref[1] + g_ref[...]
  pre = dis * tot + b_ref[...]
  z = 1.0507009873554805 * jnp.where(
      pre > 0, pre, 1.6732632423543772 * (jnp.exp(jnp.minimum(pre, 0.0)) - 1.0))
  h2 = jnp.dot(z, w_ref[...], preferred_element_type=jnp.float32)
  o_ref[...] = h2 * dis


def _tc3_body(cnt_ref, s_ref, g_ref, b_ref, o_ref):
  dis = _dis_from_cnt(cnt_ref[...])
  tot = s_ref[0] + s_ref[1] + g_ref[...]
  o = dis * tot + b_ref[...]
  m = jnp.max(o, axis=1, keepdims=True)
  e = o - m
  lse = jnp.log(jnp.sum(jnp.exp(e), axis=1, keepdims=True))
  o_ref[...] = e - lse


_cnt_spec = pl.BlockSpec((NC, _ROWS, D), lambda i: (0, i, 0))
_row_spec = pl.BlockSpec((_ROWS, D), lambda i: (i, 0))
_par_spec = pl.BlockSpec((NC, _ROWS, D), lambda i: (0, i, 0))
_w_spec = pl.BlockSpec((D, D), lambda i: (0, 0))
_b_spec = pl.BlockSpec((1, D), lambda i: (0, 0))
_out_sds = jax.ShapeDtypeStruct((N_NODES, D), jnp.float32)

_tc1a = pl.pallas_call(
    _tc1a_body, grid=(_GRID,),
    in_specs=[_row_spec, _w_spec],
    out_specs=_row_spec, out_shape=_out_sds)

_tc1b = pl.pallas_call(
    _tc1b_body, grid=(_GRID,),
    in_specs=[_cnt_spec, _row_spec],
    out_specs=_row_spec, out_shape=_out_sds)

_tc2 = pl.pallas_call(
    _tc2_body, grid=(_GRID,),
    in_specs=[_cnt_spec, _par_spec, _row_spec, _b_spec, _w_spec],
    out_specs=_row_spec, out_shape=_out_sds)

_tc3 = pl.pallas_call(
    _tc3_body, grid=(_GRID,),
    in_specs=[_cnt_spec, _par_spec, _row_spec, _b_spec],
    out_specs=_row_spec, out_shape=_out_sds)


@jax.jit
def _run(x, edge_index, W1, b1, W2, b2):
  src = edge_index[0].astype(jnp.int32)
  dst = edge_index[1].astype(jnp.int32)
  n_edges = src.shape[0]
  pad = E_PAD - n_edges
  src_f = jnp.concatenate([src, jnp.zeros((pad,), jnp.int32)])
  dst_f = jnp.concatenate([dst, jnp.full((pad,), N_NODES, jnp.int32)])
  dst_p = dst_f.reshape(NW, N_CHUNKS, CHUNK)
  src0 = src_f[:E0].reshape(NS, sum(C0_STAGES), CHUNK)
  dst0 = dst_f[:E0].reshape(NS, sum(C0_STAGES), CHUNK)
  src1 = src_f[E0:].reshape(NS, sum(C1_STAGES), CHUNK)
  dst1 = dst_f[E0:].reshape(NS, sum(C1_STAGES), CHUNK)

  zerosD = jnp.zeros((ZROWS, D), jnp.float32)
  onesD = jnp.ones((CHUNK, D), jnp.float32)

  cnt = _sc_count(dst_p, zerosD, onesD)
  h1 = _tc1a(x, W1)
  g1 = _tc1b(cnt, h1)
  s1 = _sc_segsum(g1, src0, dst0, src1, dst1, zerosD)
  g2 = _tc2(cnt, s1, g1, b1.reshape(1, D), W2)
  s2 = _sc_segsum(g2, src0, dst0, src1, dst1, zerosD)
  return _tc3(cnt, s2, g2, b2.reshape(1, D))


def kernel(x, edge_index, W1, b1, W2, b2):
  return _run(x, edge_index, W1, b1, W2, b2)
